# baseline (speedup 1.0000x reference)
import jax
import jax.numpy as jnp
import numpy as np
from jax.experimental import pallas as pl

EPS = 1e-5


def _relu(x):
    return jnp.maximum(x, 0.0)


def _bn(x, g, b):
    s = g / jnp.sqrt(1.0 + EPS)
    if x.ndim == 4:
        return x * s[None, :, None, None] + b[None, :, None, None]
    return x * s[None, :] + b[None, :]


def _conv1x1(x, W, b):
    return jnp.einsum('oc,bchw->bohw', W, x) + b[None, :, None, None]


def _set_abstraction(xyz, points, npoint, nsample, pp, perm_key):
    B, _, N = xyz.shape
    if npoint is not None:
        sel = jax.random.permutation(perm_key, N)[:npoint]
        new_xyz = xyz[:, :, sel]
    else:
        new_xyz = xyz
        npoint = N
    a = new_xyz.transpose(0, 2, 1)
    bt = xyz.transpose(0, 2, 1)
    d2 = jnp.sum(a * a, -1)[:, :, None] + jnp.sum(bt * bt, -1)[:, None, :] - 2.0 * jnp.einsum('bmd,bnd->bmn', a, bt)
    dist = jnp.sqrt(jnp.maximum(d2, 1e-12))
    _, idx = jax.lax.top_k(-dist, nsample)
    grouped = jnp.take_along_axis(bt[:, None, :, :], idx[:, :, :, None], axis=2)
    grouped = grouped - a[:, :, None, :]
    grouped_xyz = grouped.transpose(0, 3, 2, 1)
    if points is not None:
        pt = points.transpose(0, 2, 1)
        gp = jnp.take_along_axis(pt[:, None, :, :], idx[:, :, :, None], axis=2)
        gp = gp.transpose(0, 3, 2, 1)
        new_points = jnp.concatenate([grouped_xyz, gp], axis=1)
    else:
        new_points = grouped_xyz
    h = _relu(_bn(_conv1x1(new_points, pp['W1'], pp['b1']), pp['g1'], pp['be1']))
    h = _relu(_bn(_conv1x1(h, pp['W2'], pp['b2']), pp['g2'], pp['be2']))
    h = _bn(_conv1x1(h, pp['W3'], pp['b3']), pp['g3'], pp['be3'])
    new_points = jnp.max(h, axis=2)
    return new_xyz, new_points


def _head_kernel(f_ref, w1_ref, b1_ref, w2_ref, b2_ref, w3_ref, b3_ref, o_ref):
    f = f_ref[...]
    h = jnp.maximum(jnp.dot(f, w1_ref[...], preferred_element_type=jnp.float32) + b1_ref[...], 0.0)
    h = jnp.maximum(jnp.dot(h, w2_ref[...], preferred_element_type=jnp.float32) + b2_ref[...], 0.0)
    o_ref[...] = jnp.dot(h, w3_ref[...], preferred_element_type=jnp.float32) + b3_ref[...]


def _head(f, params):
    s1 = params['bn1g'] / jnp.sqrt(1.0 + EPS)
    w1 = params['fc1W'].T * s1[None, :]
    b1 = (params['fc1b'] * s1 + params['bn1b'])[None, :]
    s2 = params['bn2g'] / jnp.sqrt(1.0 + EPS)
    w2 = params['fc2W'].T * s2[None, :]
    b2 = (params['fc2b'] * s2 + params['bn2b'])[None, :]
    w3 = params['fc3W'].T
    b3 = params['fc3b'][None, :]
    return pl.pallas_call(
        _head_kernel,
        out_shape=jax.ShapeDtypeStruct((f.shape[0], 12), jnp.float32),
    )(f, w1, b1, w2, b2, w3, b3)


def kernel(x, params):
    x = x.transpose(0, 2, 1)
    xyz = x[:, :3, :]
    points = x[:, 3:, :]
    k1, k2 = jax.random.split(jax.random.key(42), 2)
    xyz1, p1 = _set_abstraction(xyz, points, 512, 32, params['sa1'], k1)
    xyz2, p2 = _set_abstraction(xyz1, p1, 128, 64, params['sa2'], k2)
    xyz3, p3 = _set_abstraction(xyz2, p2, None, 1, params['sa3'], None)
    f = jnp.max(p3, axis=2)
    return _head(f, params)


# speedup vs baseline: 1.5227x; 1.5227x over previous
import jax
import jax.numpy as jnp
import numpy as np
from jax.experimental import pallas as pl

EPS = 1e-5


def _fold(pp, i):
    s = pp['g%d' % i] / jnp.sqrt(1.0 + EPS)
    return pp['W%d' % i] * s[:, None], pp['b%d' % i] * s + pp['be%d' % i]


def _knn_idx(cent, pts, k):
    d2 = (jnp.sum(cent * cent, -1)[:, :, None]
          + jnp.sum(pts * pts, -1)[:, None, :]
          - 2.0 * jnp.einsum('bmd,bnd->bmn', cent, pts))
    _, idx = jax.lax.top_k(-d2, k)
    return idx


def _sa_stage(xyz, feats, sel, nsample, pp):
    W1, b1 = _fold(pp, 1)
    W2, b2 = _fold(pp, 2)
    W3, b3 = _fold(pp, 3)
    if feats is not None:
        pf = jnp.concatenate([xyz, feats], axis=-1)
    else:
        pf = xyz
    e = jnp.einsum('bnc,oc->bno', pf, W1)
    cent = jnp.take(xyz, sel, axis=1)
    cbias = -jnp.einsum('bmd,od->bmo', cent, W1[:, :3]) + b1[None, None, :]
    idx = _knn_idx(cent, xyz, nsample)
    g = jnp.take_along_axis(e[:, None, :, :], idx[:, :, :, None], axis=2)
    h = jnp.maximum(g + cbias[:, :, None, :], 0.0)
    h = jnp.maximum(jnp.einsum('bmkc,oc->bmko', h, W2) + b2, 0.0)
    h = jnp.einsum('bmkc,oc->bmko', h, W3) + b3
    return jnp.take(xyz, sel, axis=1), jnp.max(h, axis=2)


def _head_kernel(f_ref, w1_ref, b1_ref, w2_ref, b2_ref, w3_ref, b3_ref, o_ref):
    f = f_ref[...]
    h = jnp.maximum(jnp.dot(f, w1_ref[...], preferred_element_type=jnp.float32) + b1_ref[...], 0.0)
    h = jnp.maximum(jnp.dot(h, w2_ref[...], preferred_element_type=jnp.float32) + b2_ref[...], 0.0)
    o_ref[...] = jnp.dot(h, w3_ref[...], preferred_element_type=jnp.float32) + b3_ref[...]


def _head(f, params):
    s1 = params['bn1g'] / jnp.sqrt(1.0 + EPS)
    w1 = params['fc1W'].T * s1[None, :]
    b1 = (params['fc1b'] * s1 + params['bn1b'])[None, :]
    s2 = params['bn2g'] / jnp.sqrt(1.0 + EPS)
    w2 = params['fc2W'].T * s2[None, :]
    b2 = (params['fc2b'] * s2 + params['bn2b'])[None, :]
    w3 = params['fc3W'].T
    b3 = params['fc3b'][None, :]
    return pl.pallas_call(
        _head_kernel,
        out_shape=jax.ShapeDtypeStruct((f.shape[0], 12), jnp.float32),
    )(f, w1, b1, w2, b2, w3, b3)


def kernel(x, params):
    xyz = x[:, :, :3]
    feats = x[:, :, 3:]
    k1, k2 = jax.random.split(jax.random.key(42), 2)
    sel1 = jax.random.permutation(k1, 4096)[:512]
    sel2 = jax.random.permutation(k2, 512)[:128]

    xyz1, p1 = _sa_stage(xyz, feats, sel1, 32, params['sa1'])
    xyz2, p2 = _sa_stage(xyz1, p1, sel2, 64, params['sa2'])

    pp = params['sa3']
    W1, b1 = _fold(pp, 1)
    W2, b2 = _fold(pp, 2)
    W3, b3 = _fold(pp, 3)
    h = jnp.maximum(jnp.einsum('bnc,oc->bno', p2, W1[:, 3:]) + b1, 0.0)
    h = jnp.maximum(jnp.einsum('bnc,oc->bno', h, W2) + b2, 0.0)
    h = jnp.einsum('bnc,oc->bno', h, W3) + b3
    f = jnp.max(h, axis=1)
    return _head(f, params)


# speedup vs baseline: 2.6231x; 1.7226x over previous
import jax
import jax.numpy as jnp
import numpy as np
from jax.experimental import pallas as pl
from jax.experimental.pallas import tpu as pltpu

EPS = 1e-5


def _fold(pp, i):
    s = pp['g%d' % i] / jnp.sqrt(1.0 + EPS)
    return pp['W%d' % i] * s[:, None], pp['b%d' % i] * s + pp['be%d' % i]


def _knn_kernel(k, cent_ref, pts_ref, idx_ref, d_ref):
    M, N = d_ref.shape
    c = cent_ref[...]
    cx, cy, cz = c[:, 0:1], c[:, 1:2], c[:, 2:3]
    px, py, pz = pts_ref[0:1, :], pts_ref[1:2, :], pts_ref[2:3, :]
    cn2 = cx * cx + cy * cy + cz * cz
    pn2 = px * px + py * py + pz * pz
    d_ref[...] = (cn2 + pn2) - 2.0 * (cx * px + cy * py + cz * pz)

    lane = jax.lax.broadcasted_iota(jnp.int32, (M, N), 1)
    olane = jax.lax.broadcasted_iota(jnp.int32, (M, k), 1)
    big_i = jnp.int32(2 ** 30)
    inf = jnp.float32(np.inf)

    def body(it, _):
        d = d_ref[...]
        rowmin = jnp.min(d, axis=1, keepdims=True)
        cand = jnp.where(d == rowmin, lane, big_i)
        amin = jnp.min(cand, axis=1, keepdims=True)
        idx_ref[...] = jnp.where(olane == it, amin, idx_ref[...])
        d_ref[...] = jnp.where(cand == amin, inf, d)
        return 0

    jax.lax.fori_loop(0, k, body, 0)


def _knn_idx(cent, pts, k):
    import functools
    B, M, _ = cent.shape
    N = pts.shape[1]
    return pl.pallas_call(
        functools.partial(_knn_kernel, k),
        grid=(B,),
        in_specs=[
            pl.BlockSpec((None, M, 3), lambda b: (b, 0, 0)),
            pl.BlockSpec((None, 3, N), lambda b: (b, 0, 0)),
        ],
        out_specs=pl.BlockSpec((None, M, k), lambda b: (b, 0, 0)),
        out_shape=jax.ShapeDtypeStruct((B, M, k), jnp.int32),
        scratch_shapes=[pltpu.VMEM((M, N), jnp.float32)],
    )(cent, pts.transpose(0, 2, 1))


def _sa_stage(xyz, feats, sel, nsample, pp):
    W1, b1 = _fold(pp, 1)
    W2, b2 = _fold(pp, 2)
    W3, b3 = _fold(pp, 3)
    if feats is not None:
        pf = jnp.concatenate([xyz, feats], axis=-1)
    else:
        pf = xyz
    e = jnp.einsum('bnc,oc->bno', pf, W1)
    cent = jnp.take(xyz, sel, axis=1)
    cbias = -jnp.einsum('bmd,od->bmo', cent, W1[:, :3]) + b1[None, None, :]
    idx = _knn_idx(cent, xyz, nsample)
    g = jnp.take_along_axis(e[:, None, :, :], idx[:, :, :, None], axis=2)
    h = jnp.maximum(g + cbias[:, :, None, :], 0.0)
    h = jnp.maximum(jnp.einsum('bmkc,oc->bmko', h, W2) + b2, 0.0)
    h = jnp.einsum('bmkc,oc->bmko', h, W3) + b3
    return jnp.take(xyz, sel, axis=1), jnp.max(h, axis=2)


def _head_kernel(f_ref, w1_ref, b1_ref, w2_ref, b2_ref, w3_ref, b3_ref, o_ref):
    f = f_ref[...]
    h = jnp.maximum(jnp.dot(f, w1_ref[...], preferred_element_type=jnp.float32) + b1_ref[...], 0.0)
    h = jnp.maximum(jnp.dot(h, w2_ref[...], preferred_element_type=jnp.float32) + b2_ref[...], 0.0)
    o_ref[...] = jnp.dot(h, w3_ref[...], preferred_element_type=jnp.float32) + b3_ref[...]


def _head(f, params):
    s1 = params['bn1g'] / jnp.sqrt(1.0 + EPS)
    w1 = params['fc1W'].T * s1[None, :]
    b1 = (params['fc1b'] * s1 + params['bn1b'])[None, :]
    s2 = params['bn2g'] / jnp.sqrt(1.0 + EPS)
    w2 = params['fc2W'].T * s2[None, :]
    b2 = (params['fc2b'] * s2 + params['bn2b'])[None, :]
    w3 = params['fc3W'].T
    b3 = params['fc3b'][None, :]
    return pl.pallas_call(
        _head_kernel,
        out_shape=jax.ShapeDtypeStruct((f.shape[0], 12), jnp.float32),
    )(f, w1, b1, w2, b2, w3, b3)


def kernel(x, params):
    xyz = x[:, :, :3]
    feats = x[:, :, 3:]
    k1, k2 = jax.random.split(jax.random.key(42), 2)
    sel1 = jax.random.permutation(k1, 4096)[:512]
    sel2 = jax.random.permutation(k2, 512)[:128]

    xyz1, p1 = _sa_stage(xyz, feats, sel1, 32, params['sa1'])
    xyz2, p2 = _sa_stage(xyz1, p1, sel2, 64, params['sa2'])

    pp = params['sa3']
    W1, b1 = _fold(pp, 1)
    W2, b2 = _fold(pp, 2)
    W3, b3 = _fold(pp, 3)
    h = jnp.maximum(jnp.einsum('bnc,oc->bno', p2, W1[:, 3:]) + b1, 0.0)
    h = jnp.maximum(jnp.einsum('bnc,oc->bno', h, W2) + b2, 0.0)
    h = jnp.einsum('bnc,oc->bno', h, W3) + b3
    f = jnp.max(h, axis=1)
    return _head(f, params)
